# baseline (speedup 1.0000x reference)
import functools
import jax
import jax.numpy as jnp
from jax import lax
from jax.experimental import pallas as pl
from jax.experimental.pallas import tpu as pltpu, tpu_sc as plsc

B = 32768
D0, D1, D2 = 16, 56, 56
DTOT = D0 + D1 + D2
V = 512

NC, NS = 2, 16
REPL = 8
NW = NC * NS
B_PER_W = B // NW
CHUNK = 128
N_CHUNK = B_PER_W // CHUNK
DEPTH = 4


def _sc_body(idx_hbm, t0_hbm, t1_hbm, t2_hbm, out_hbm,
             idx_v, combs, sem_in, sems_g, sems_a, sems_w):
    wid = lax.axis_index("s") * NC + lax.axis_index("c")
    pltpu.async_copy(idx_hbm.at[wid], idx_v, sem_in).wait()

    g_pend = [None] * DEPTH
    a_pend = [None] * DEPTH
    w_pend = [None] * DEPTH

    def fire_g0(c):
        p = c % DEPTH
        if w_pend[p] is not None:
            w_pend[p].wait()
        g_pend[p] = pltpu.async_copy(
            t0_hbm.at[idx_v.at[0, c]], combs[p], sems_g[p])

    def fire_adds(c):
        p = c % DEPTH
        g_pend[p].wait()
        a_pend[p] = (
            pltpu.async_copy(t1_hbm.at[idx_v.at[1, c]], combs[p],
                             sems_a[p], add=True),
            pltpu.async_copy(t2_hbm.at[idx_v.at[2, c]], combs[p],
                             sems_a[p], add=True),
        )

    def fire_write(c):
        p = c % DEPTH
        a_pend[p][0].wait()
        a_pend[p][1].wait()
        base = wid * B_PER_W + c * CHUNK
        w_pend[p] = pltpu.async_copy(
            combs[p], out_hbm.at[pl.ds(base, CHUNK), :], sems_w[p])

    fire_g0(0)
    fire_g0(1)
    fire_adds(0)
    for c in range(2, N_CHUNK):
        fire_g0(c)
        fire_adds(c - 1)
        fire_write(c - 2)
    fire_adds(N_CHUNK - 1)
    fire_write(N_CHUNK - 2)
    fire_write(N_CHUNK - 1)
    for p in range(DEPTH):
        if w_pend[p] is not None:
            w_pend[p].wait()


@jax.jit
def _rope_gather(idx, t0, t1, t2):
    mesh = plsc.VectorSubcoreMesh(core_axis_name="c", subcore_axis_name="s",
                                  num_cores=NC)

    def body(idx_hbm, t0_hbm, t1_hbm, t2_hbm, out_hbm, idx_v,
             c0, c1, c2, c3, sem_in,
             g0, g1, g2, g3, a0, a1, a2, a3, w0, w1, w2, w3):
        _sc_body(idx_hbm, t0_hbm, t1_hbm, t2_hbm, out_hbm, idx_v,
                 (c0, c1, c2, c3), sem_in,
                 (g0, g1, g2, g3), (a0, a1, a2, a3), (w0, w1, w2, w3))

    f = pl.kernel(
        body,
        out_type=jax.ShapeDtypeStruct((B, DTOT), jnp.float32),
        mesh=mesh,
        scratch_types=[
            pltpu.VMEM((3, N_CHUNK, CHUNK), jnp.int32),
        ] + [pltpu.VMEM((CHUNK, DTOT), jnp.float32)] * DEPTH
          + [pltpu.SemaphoreType.DMA] * (1 + 3 * DEPTH),
        compiler_params=pltpu.CompilerParams(
            disable_bounds_checks=True,
            skip_device_barrier=True,
        ),
    )
    return f(idx, t0, t1, t2)


def kernel(ids, freqs_0, freqs_1, freqs_2):
    idx = ids.astype(jnp.int32).T.reshape(3, NW, N_CHUNK, CHUNK)
    idx = idx.transpose(1, 0, 2, 3)
    repl_off = (jnp.arange(NW, dtype=jnp.int32) % REPL) * V
    idx = idx + repl_off[:, None, None, None]
    t0 = freqs_0[:V].reshape(V, D0)
    t1 = freqs_1[:V].reshape(V, D1)
    t2 = freqs_2[:V].reshape(V, D2)
    z0 = jnp.zeros((V, D0), jnp.float32)
    z1 = jnp.zeros((V, D1), jnp.float32)
    z2 = jnp.zeros((V, D2), jnp.float32)
    t0p = jnp.tile(jnp.concatenate([t0, z1, z2], axis=1), (REPL, 1))
    t1p = jnp.tile(jnp.concatenate([z0, t1, z2], axis=1), (REPL, 1))
    t2p = jnp.tile(jnp.concatenate([z0, z1, t2], axis=1), (REPL, 1))
    out = _rope_gather(idx, t0p, t1p, t2p)
    return out.reshape(B, DTOT // 2, 2)


# speedup vs baseline: 1.0198x; 1.0198x over previous
import functools
import jax
import jax.numpy as jnp
from jax import lax
from jax.experimental import pallas as pl
from jax.experimental.pallas import tpu as pltpu, tpu_sc as plsc

B = 32768
D0, D1, D2 = 16, 56, 56
DTOT = D0 + D1 + D2
V = 512

NC, NS = 2, 16
NW = NC * NS
B_PER_W = B // NW
CHUNK = 128
N_CHUNK = B_PER_W // CHUNK
DEPTH = 4
REPL = 8


def _sc_body(idx_hbm, t0_hbm, t1_hbm, t2_hbm, out_hbm,
             idx_v, t0_v, combs, sem_in, sems_g, sems_a, sems_w):
    wid = lax.axis_index("s") * NC + lax.axis_index("c")
    cp_i = pltpu.async_copy(idx_hbm.at[wid], idx_v, sem_in)
    cp_t = pltpu.async_copy(t0_hbm, t0_v, sem_in)
    cp_i.wait()
    cp_t.wait()

    g_pend = [None] * DEPTH
    a_pend = [None] * DEPTH
    w_pend = [None] * DEPTH

    def fire_g1(c):
        p = c % DEPTH
        if w_pend[p] is not None:
            w_pend[p].wait()
        g_pend[p] = pltpu.async_copy(
            t1_hbm.at[idx_v.at[1, c]], combs[p], sems_g[p])

    def fire_g2(c):
        p = c % DEPTH
        g_pend[p].wait()
        a_pend[p] = pltpu.async_copy(
            t2_hbm.at[idx_v.at[2, c]], combs[p], sems_a[p], add=True)

    def fill_and_write(c):
        p = c % DEPTH
        a_pend[p].wait()
        comb = combs[p]

        def group_body(g, carry):
            a_vec = idx_v[0, c, pl.ds(g * 16, 16)]
            for l in range(16):
                a = a_vec[l]
                comb[g * 16 + l, pl.ds(0, D0)] = t0_v[pl.ds(a * D0, D0)]
            return carry

        lax.fori_loop(0, CHUNK // 16, group_body, 0)
        base = wid * B_PER_W + c * CHUNK
        w_pend[p] = pltpu.async_copy(
            comb, out_hbm.at[pl.ds(base, CHUNK), :], sems_w[p])

    fire_g1(0)
    fire_g1(1)
    fire_g2(0)
    for c in range(2, N_CHUNK):
        fire_g1(c)
        fire_g2(c - 1)
        fill_and_write(c - 2)
    fire_g2(N_CHUNK - 1)
    fill_and_write(N_CHUNK - 2)
    fill_and_write(N_CHUNK - 1)
    for p in range(DEPTH):
        if w_pend[p] is not None:
            w_pend[p].wait()


@jax.jit
def _rope_gather(idx, t0, t1, t2):
    mesh = plsc.VectorSubcoreMesh(core_axis_name="c", subcore_axis_name="s",
                                  num_cores=NC)

    def body(idx_hbm, t0_hbm, t1_hbm, t2_hbm, out_hbm, idx_v, t0_v,
             c0, c1, c2, c3, sem_in,
             g0, g1, g2, g3, a0, a1, a2, a3, w0, w1, w2, w3):
        _sc_body(idx_hbm, t0_hbm, t1_hbm, t2_hbm, out_hbm, idx_v, t0_v,
                 (c0, c1, c2, c3), sem_in,
                 (g0, g1, g2, g3), (a0, a1, a2, a3), (w0, w1, w2, w3))

    f = pl.kernel(
        body,
        out_type=jax.ShapeDtypeStruct((B, DTOT), jnp.float32),
        mesh=mesh,
        scratch_types=[
            pltpu.VMEM((3, N_CHUNK, CHUNK), jnp.int32),
            pltpu.VMEM((V * D0,), jnp.float32),
        ] + [pltpu.VMEM((CHUNK, DTOT), jnp.float32)] * DEPTH
          + [pltpu.SemaphoreType.DMA] * (1 + 3 * DEPTH),
        compiler_params=pltpu.CompilerParams(
            disable_bounds_checks=True,
            skip_device_barrier=True,
        ),
    )
    return f(idx, t0, t1, t2)


def kernel(ids, freqs_0, freqs_1, freqs_2):
    idx = ids.astype(jnp.int32).T.reshape(3, NW, N_CHUNK, CHUNK)
    idx = idx.transpose(1, 0, 2, 3)
    repl_off = (jnp.arange(NW, dtype=jnp.int32) % REPL) * V
    idx = idx.at[:, 1:3, :, :].add(repl_off[:, None, None, None])
    t0 = freqs_0[:V].reshape(V * D0)
    t1 = freqs_1[:V].reshape(V, D1)
    t2 = freqs_2[:V].reshape(V, D2)
    z0 = jnp.zeros((V, D0), jnp.float32)
    z1 = jnp.zeros((V, D1), jnp.float32)
    z2 = jnp.zeros((V, D2), jnp.float32)
    t1p = jnp.tile(jnp.concatenate([z0, t1, z2], axis=1), (REPL, 1))
    t2p = jnp.tile(jnp.concatenate([z0, z1, t2], axis=1), (REPL, 1))
    out = _rope_gather(idx, t0, t1p, t2p)
    return out.reshape(B, DTOT // 2, 2)


# speedup vs baseline: 1.0789x; 1.0580x over previous
import functools
import jax
import jax.numpy as jnp
from jax import lax
from jax.experimental import pallas as pl
from jax.experimental.pallas import tpu as pltpu, tpu_sc as plsc

B = 32768
D0, D1, D2 = 16, 56, 56
DTOT = D0 + D1 + D2
V = 512

NC, NS = 2, 16
NW = NC * NS
B_PER_W = B // NW
CHUNK = 128
N_CHUNK = B_PER_W // CHUNK
DEPTH = 2
REPL = 8
SKEW = CHUNK
GROUP = 4


def _sc_body(idx_hbm, t0_hbm, t1_hbm, t2_hbm, out_hbm,
             idx_v, t0_v, combT, combs, sem_in, sems_g, sems_a, sem_w):
    wid = lax.axis_index("s") * NC + lax.axis_index("c")
    cp_i = pltpu.async_copy(idx_hbm.at[wid], idx_v, sem_in)
    cp_t = pltpu.async_copy(t0_hbm, t0_v, sem_in)
    cp_i.wait()
    cp_t.wait()

    g_pend = [None] * DEPTH
    a_pend = [None] * DEPTH
    w_pend = [None]

    def fire_g1(c):
        p = c % DEPTH
        g_pend[p] = pltpu.async_copy(
            t1_hbm.at[idx_v.at[1, c]], combs[p].at[:, pl.ds(0, DTOT)],
            sems_g[p])

    def fire_g2(c):
        p = c % DEPTH
        g_pend[p].wait()
        a_pend[p] = pltpu.async_copy(
            t2_hbm.at[idx_v.at[2, c]], combs[p].at[:, pl.ds(0, DTOT)],
            sems_a[p], add=True)

    def finish(c):
        p = c % DEPTH
        a_pend[p].wait()
        comb = combs[p]

        def group_body(g, carry):
            a_vec = idx_v[0, c, pl.ds(g * 16, 16)]
            for l in range(16):
                a = a_vec[l]
                comb[g * 16 + l, pl.ds(0, D0)] = t0_v[pl.ds(a * D0, D0)]
            return carry

        lax.fori_loop(0, CHUNK // 16, group_body, 0)

        if c % GROUP == 0 and w_pend[0] is not None:
            w_pend[0].wait()
            w_pend[0] = None

        u_base = (c % GROUP) * 2
        lanes = lax.iota(jnp.int32, 16)

        def blk_body(b, carry):
            tb = (b // (DTOT // 16)) * 16
            rb = (b % (DTOT // 16)) * 16
            ti = tb + lanes
            for d in range(16):
                r = rb + ((lanes + d) & 15)
                vals = plsc.load_gather(comb, [ti, r])
                plsc.store_scatter(
                    combT, [r >> 1, u_base + (r & 1), ti], vals)
            return carry

        lax.fori_loop(0, (CHUNK // 16) * (DTOT // 16), blk_body, 0)

    def fire_write(q):
        w_pend[0] = pltpu.async_copy(
            combT, out_hbm.at[:, pl.ds(wid * 16 + 8 * q, 8), :], sem_w)

    def maybe_write_after(c):
        if c % GROUP == GROUP - 1:
            fire_write(c // GROUP)

    fire_g1(0)
    fire_g1(1)
    fire_g2(0)
    for c in range(N_CHUNK):
        if c + 1 < N_CHUNK:
            fire_g2(c + 1)
        finish(c)
        maybe_write_after(c)
        if c + 2 < N_CHUNK:
            fire_g1(c + 2)
    if w_pend[0] is not None:
        w_pend[0].wait()


@jax.jit
def _rope_gather(idx, t0, t1, t2):
    mesh = plsc.VectorSubcoreMesh(core_axis_name="c", subcore_axis_name="s",
                                  num_cores=NC)

    def body(idx_hbm, t0_hbm, t1_hbm, t2_hbm, out_hbm, idx_v, t0_v, combT,
             c0, c1, sem_in, g0, g1, a0, a1, w0):
        _sc_body(idx_hbm, t0_hbm, t1_hbm, t2_hbm, out_hbm, idx_v, t0_v,
                 combT, (c0, c1), sem_in, (g0, g1), (a0, a1), w0)

    f = pl.kernel(
        body,
        out_type=jax.ShapeDtypeStruct((DTOT // 2, B // CHUNK * 2, CHUNK),
                                      jnp.float32),
        mesh=mesh,
        scratch_types=[
            pltpu.VMEM((3, N_CHUNK, CHUNK), jnp.int32),
            pltpu.VMEM((V * D0,), jnp.float32),
            pltpu.VMEM((DTOT // 2, 2 * GROUP, CHUNK), jnp.float32),
        ] + [pltpu.VMEM((CHUNK, SKEW), jnp.float32)] * DEPTH
          + [pltpu.SemaphoreType.DMA] * (1 + 2 * DEPTH + 1),
        compiler_params=pltpu.CompilerParams(
            disable_bounds_checks=True,
            skip_device_barrier=True,
            needs_layout_passes=False,
        ),
    )
    return f(idx, t0, t1, t2)


def kernel(ids, freqs_0, freqs_1, freqs_2):
    idx = ids.astype(jnp.int32).T.reshape(3, NW, N_CHUNK, CHUNK)
    idx = idx.transpose(1, 0, 2, 3)
    repl_off = (jnp.arange(NW, dtype=jnp.int32) % REPL) * V
    idx = idx.at[:, 1:3, :, :].add(repl_off[:, None, None, None])
    t0 = freqs_0[:V].reshape(V * D0)
    t1 = freqs_1[:V].reshape(V, D1)
    t2 = freqs_2[:V].reshape(V, D2)
    z0 = jnp.zeros((V, D0), jnp.float32)
    z1 = jnp.zeros((V, D1), jnp.float32)
    z2 = jnp.zeros((V, D2), jnp.float32)
    t1p = jnp.tile(jnp.concatenate([z0, t1, z2], axis=1), (REPL, 1))
    t2p = jnp.tile(jnp.concatenate([z0, z1, t2], axis=1), (REPL, 1))
    out = _rope_gather(idx, t0, t1p, t2p)
    out = out.reshape(64, 256, 2, 128).transpose(1, 3, 0, 2)
    return out.reshape(B, DTOT // 2, 2)
